# baseline (speedup 1.0000x reference)
import jax
import jax.numpy as jnp
from jax import lax
from jax.experimental import pallas as pl
from jax.experimental.pallas import tpu as pltpu
from jax.experimental.pallas import tpu_sc as plsc

B = 16384
D = 32
NC = 2
NS = 16
NW = NC * NS
BPW = B // NW
G = 8
NG = BPW // G

BLK = 2048


def _gather_one_table(idx_v, tabT, winA, winB, stageT, sem):
    r0 = lax.iota(jnp.int32, 16)

    def issue(idv16, half, buf):
        for l in range(G):
            w0 = pl.multiple_of((idv16[half * G + l] >> 7) * 128, 128)
            pltpu.async_copy(tabT.at[:, pl.ds(w0, 128)], buf.at[l], sem)

    def drain(buf):
        for l in range(G):
            pltpu.make_async_copy(tabT.at[:, pl.ds(0, 128)],
                                  buf.at[l], sem).wait()

    def extract(k, idv16, half, buf):
        for l in range(G):
            colv = jnp.full((16,), idv16[half * G + l] & 127, jnp.int32)
            g0 = plsc.load_gather(buf.at[l], [r0, colv])
            g1 = plsc.load_gather(buf.at[l], [r0 + 16, colv])
            jcol = jnp.full((16,), k * 16 + half * G + l, jnp.int32)
            plsc.store_scatter(stageT, [r0, jcol], g0)
            plsc.store_scatter(stageT, [r0 + 16, jcol], g1)

    issue(idx_v[pl.ds(0, 16)], 0, winA)

    def body(k, _):
        idvk = idx_v[pl.ds(k * 16, 16)]
        issue(idvk, 1, winB)
        drain(winA)
        extract(k, idvk, 0, winA)
        idvn = idx_v[pl.ds(jnp.minimum((k + 1) * 16, BPW - 16), 16)]
        issue(idvn, 0, winA)
        drain(winB)
        extract(k, idvk, 1, winB)
        return 0

    lax.fori_loop(0, BPW // 16, body, 0)
    drain(winA)


def _sc_gather(cust_ids, prod_ids, cembT, pembT,
               ceT_out, peT_out,
               cidx_v, pidx_v, winA, winB, stageT, sem):
    wid = lax.axis_index("s") * NC + lax.axis_index("c")
    base = wid * BPW
    pltpu.sync_copy(cust_ids.at[pl.ds(base, BPW)], cidx_v)
    pltpu.sync_copy(prod_ids.at[pl.ds(base, BPW)], pidx_v)

    _gather_one_table(cidx_v, cembT, winA, winB, stageT, sem)
    pltpu.sync_copy(stageT, ceT_out.at[:, pl.ds(base, BPW)])
    _gather_one_table(pidx_v, pembT, winA, winB, stageT, sem)
    pltpu.sync_copy(stageT, peT_out.at[:, pl.ds(base, BPW)])


def _sc_bias(cust_ids, prod_ids, cbias, pbias, cb_out, pb_out,
             cidx_v, pidx_v, cb_v, pb_v, bsem):
    wid = lax.axis_index("s") * NC + lax.axis_index("c")
    base = wid * BPW
    pltpu.sync_copy(cust_ids.at[pl.ds(base, BPW)], cidx_v)
    pltpu.sync_copy(prod_ids.at[pl.ds(base, BPW)], pidx_v)
    pltpu.async_copy(cbias.at[cidx_v], cb_v, bsem)
    pltpu.async_copy(pbias.at[pidx_v], pb_v, bsem)
    pltpu.make_async_copy(cbias.at[pl.ds(0, BPW)], cb_v, bsem).wait()
    pltpu.make_async_copy(pbias.at[pl.ds(0, BPW)], pb_v, bsem).wait()
    pltpu.sync_copy(cb_v, cb_out.at[pl.ds(base, BPW)])
    pltpu.sync_copy(pb_v, pb_out.at[pl.ds(base, BPW)])


def _dense_body(ceT_ref, peT_ref, cb_ref, pb_ref, w1cT_ref, w1pT_ref, b1_ref,
                w2T_ref, b2_ref, w3_ref, const_ref, out_ref):
    ceT = ceT_ref[...]
    peT = peT_ref[...]
    mf = jnp.sum(ceT * peT, axis=0, keepdims=True)
    h1 = jnp.maximum(
        jnp.dot(w1cT_ref[...], ceT, preferred_element_type=jnp.float32)
        + jnp.dot(w1pT_ref[...], peT, preferred_element_type=jnp.float32)
        + b1_ref[...], 0.0)
    h2 = jnp.maximum(
        jnp.dot(w2T_ref[...], h1, preferred_element_type=jnp.float32)
        + b2_ref[...], 0.0)
    mlp = jnp.sum(h2 * w3_ref[...], axis=0, keepdims=True)
    logit = (0.6 * mf + 0.4 * mlp + cb_ref[...] + pb_ref[...]
             + const_ref[...])
    out_ref[...] = jax.nn.sigmoid(logit)


def kernel(customer_ids, product_ids, cust_emb, prod_emb, cust_bias,
           prod_bias, global_bias, W1, b1, W2, b2, W3, b3):
    cids = customer_ids.astype(jnp.int32)
    pids = product_ids.astype(jnp.int32)
    cembT = cust_emb.T
    pembT = prod_emb.T
    cbias = cust_bias.reshape(-1)
    pbias = prod_bias.reshape(-1)

    mesh = plsc.VectorSubcoreMesh(
        core_axis_name="c", subcore_axis_name="s",
        num_cores=NC, num_subcores=NS)
    sc_call = pl.kernel(
        _sc_gather,
        out_type=[
            jax.ShapeDtypeStruct((D, B), jnp.float32),
            jax.ShapeDtypeStruct((D, B), jnp.float32),
        ],
        mesh=mesh,
        scratch_types=[
            pltpu.VMEM((BPW,), jnp.int32),
            pltpu.VMEM((BPW,), jnp.int32),
            pltpu.VMEM((G, D, 128), jnp.float32),
            pltpu.VMEM((G, D, 128), jnp.float32),
            pltpu.VMEM((D, BPW), jnp.float32),
            pltpu.SemaphoreType.DMA,
        ],
        compiler_params=pltpu.CompilerParams(use_tc_tiling_on_sc=True,
                                             needs_layout_passes=False),
    )
    ceT, peT = sc_call(cids, pids, cembT, pembT)

    bias_call = pl.kernel(
        _sc_bias,
        out_type=[
            jax.ShapeDtypeStruct((B,), jnp.float32),
            jax.ShapeDtypeStruct((B,), jnp.float32),
        ],
        mesh=mesh,
        scratch_types=[
            pltpu.VMEM((BPW,), jnp.int32),
            pltpu.VMEM((BPW,), jnp.int32),
            pltpu.VMEM((BPW,), jnp.float32),
            pltpu.VMEM((BPW,), jnp.float32),
            pltpu.SemaphoreType.DMA,
        ],
        compiler_params=pltpu.CompilerParams(use_tc_tiling_on_sc=True,
                                             needs_layout_passes=False),
    )
    cb, pb = bias_call(cids, pids, cbias, pbias)

    w1cT = W1[:D, :].T
    w1pT = W1[D:, :].T
    const = (0.4 * b3 + global_bias).reshape(1, 1)

    grid = (B // BLK,)
    out = pl.pallas_call(
        _dense_body,
        grid=grid,
        in_specs=[
            pl.BlockSpec((D, BLK), lambda i: (0, i)),
            pl.BlockSpec((D, BLK), lambda i: (0, i)),
            pl.BlockSpec((1, BLK), lambda i: (0, i)),
            pl.BlockSpec((1, BLK), lambda i: (0, i)),
            pl.BlockSpec((64, D), lambda i: (0, 0)),
            pl.BlockSpec((64, D), lambda i: (0, 0)),
            pl.BlockSpec((64, 1), lambda i: (0, 0)),
            pl.BlockSpec((D, 64), lambda i: (0, 0)),
            pl.BlockSpec((D, 1), lambda i: (0, 0)),
            pl.BlockSpec((D, 1), lambda i: (0, 0)),
            pl.BlockSpec((1, 1), lambda i: (0, 0)),
        ],
        out_specs=pl.BlockSpec((1, BLK), lambda i: (0, i)),
        out_shape=jax.ShapeDtypeStruct((1, B), jnp.float32),
    )(ceT, peT, cb.reshape(1, B), pb.reshape(1, B), w1cT, w1pT,
      b1.reshape(64, 1), W2.T, b2.reshape(D, 1), W3, const)
    return out.reshape(B)


# speedup vs baseline: 1.0385x; 1.0385x over previous
import jax
import jax.numpy as jnp
from jax import lax
from jax.experimental import pallas as pl
from jax.experimental.pallas import tpu as pltpu
from jax.experimental.pallas import tpu_sc as plsc

B = 16384
D = 32
NC = 2
NS = 16
NW = NC * NS
BPW = B // NW
G = 4

BLK = 2048


def _gather_one_table(idx_v, tabT, bufs, stageT, sem):
    r0 = lax.iota(jnp.int32, 16)

    def issue_q(idv16, q, buf):
        for l in range(G):
            w0 = pl.multiple_of((idv16[G * q + l] >> 7) * 128, 128)
            pltpu.async_copy(tabT.at[:, pl.ds(w0, 128)], buf.at[l], sem)

    def drain_q(buf):
        for l in range(G):
            pltpu.make_async_copy(tabT.at[:, pl.ds(0, 128)],
                                  buf.at[l], sem).wait()

    def extract_q(k, idv16, q, buf):
        for l in range(G):
            colv = jnp.full((16,), idv16[G * q + l] & 127, jnp.int32)
            g0 = plsc.load_gather(buf.at[l], [r0, colv])
            g1 = plsc.load_gather(buf.at[l], [r0 + 16, colv])
            jcol = jnp.full((16,), k * 16 + G * q + l, jnp.int32)
            plsc.store_scatter(stageT, [r0, jcol], g0)
            plsc.store_scatter(stageT, [r0 + 16, jcol], g1)

    b0, b1, b2, b3 = bufs
    idv_first = idx_v[pl.ds(0, 16)]
    issue_q(idv_first, 0, b0)
    issue_q(idv_first, 1, b1)
    issue_q(idv_first, 2, b2)

    def body(k, _):
        idvk = idx_v[pl.ds(k * 16, 16)]
        idvn = idx_v[pl.ds(jnp.minimum((k + 1) * 16, BPW - 16), 16)]
        issue_q(idvk, 3, b3)
        drain_q(b0)
        extract_q(k, idvk, 0, b0)
        issue_q(idvn, 0, b0)
        drain_q(b1)
        extract_q(k, idvk, 1, b1)
        issue_q(idvn, 1, b1)
        drain_q(b2)
        extract_q(k, idvk, 2, b2)
        issue_q(idvn, 2, b2)
        drain_q(b3)
        extract_q(k, idvk, 3, b3)
        return 0

    lax.fori_loop(0, BPW // 16, body, 0)
    drain_q(b0)
    drain_q(b1)
    drain_q(b2)


def _sc_gather(cust_ids, prod_ids, cembT, pembT, cbias, pbias,
               ceT_out, peT_out, cb_out, pb_out,
               cidx_v, pidx_v, b0, b1, b2, b3, stageT, cb_v, pb_v,
               sem, bsem):
    wid = lax.axis_index("s") * NC + lax.axis_index("c")
    base = wid * BPW
    pltpu.sync_copy(cust_ids.at[pl.ds(base, BPW)], cidx_v)
    pltpu.sync_copy(prod_ids.at[pl.ds(base, BPW)], pidx_v)

    pltpu.async_copy(cbias.at[cidx_v], cb_v, bsem)
    pltpu.async_copy(pbias.at[pidx_v], pb_v, bsem)

    _gather_one_table(cidx_v, cembT, (b0, b1, b2, b3), stageT, sem)
    pltpu.sync_copy(stageT, ceT_out.at[:, pl.ds(base, BPW)])
    _gather_one_table(pidx_v, pembT, (b0, b1, b2, b3), stageT, sem)
    pltpu.sync_copy(stageT, peT_out.at[:, pl.ds(base, BPW)])

    pltpu.make_async_copy(cbias.at[pl.ds(0, BPW)], cb_v, bsem).wait()
    pltpu.make_async_copy(pbias.at[pl.ds(0, BPW)], pb_v, bsem).wait()
    pltpu.sync_copy(cb_v, cb_out.at[pl.ds(base, BPW)])
    pltpu.sync_copy(pb_v, pb_out.at[pl.ds(base, BPW)])


def _dense_body(ceT_ref, peT_ref, cb_ref, pb_ref, w1cT_ref, w1pT_ref, b1_ref,
                w2T_ref, b2_ref, w3_ref, const_ref, out_ref):
    ceT = ceT_ref[...]
    peT = peT_ref[...]
    mf = jnp.sum(ceT * peT, axis=0, keepdims=True)
    h1 = jnp.maximum(
        jnp.dot(w1cT_ref[...], ceT, preferred_element_type=jnp.float32)
        + jnp.dot(w1pT_ref[...], peT, preferred_element_type=jnp.float32)
        + b1_ref[...], 0.0)
    h2 = jnp.maximum(
        jnp.dot(w2T_ref[...], h1, preferred_element_type=jnp.float32)
        + b2_ref[...], 0.0)
    mlp = jnp.sum(h2 * w3_ref[...], axis=0, keepdims=True)
    logit = (0.6 * mf + 0.4 * mlp + cb_ref[...] + pb_ref[...]
             + const_ref[...])
    out_ref[...] = jax.nn.sigmoid(logit)


def kernel(customer_ids, product_ids, cust_emb, prod_emb, cust_bias,
           prod_bias, global_bias, W1, b1, W2, b2, W3, b3):
    cids = customer_ids.astype(jnp.int32)
    pids = product_ids.astype(jnp.int32)
    cembT = cust_emb.T
    pembT = prod_emb.T
    cbias = cust_bias.reshape(-1)
    pbias = prod_bias.reshape(-1)

    mesh = plsc.VectorSubcoreMesh(
        core_axis_name="c", subcore_axis_name="s",
        num_cores=NC, num_subcores=NS)
    sc_call = pl.kernel(
        _sc_gather,
        out_type=[
            jax.ShapeDtypeStruct((D, B), jnp.float32),
            jax.ShapeDtypeStruct((D, B), jnp.float32),
            jax.ShapeDtypeStruct((B,), jnp.float32),
            jax.ShapeDtypeStruct((B,), jnp.float32),
        ],
        mesh=mesh,
        scratch_types=[
            pltpu.VMEM((BPW,), jnp.int32),
            pltpu.VMEM((BPW,), jnp.int32),
            pltpu.VMEM((G, D, 128), jnp.float32),
            pltpu.VMEM((G, D, 128), jnp.float32),
            pltpu.VMEM((G, D, 128), jnp.float32),
            pltpu.VMEM((G, D, 128), jnp.float32),
            pltpu.VMEM((D, BPW), jnp.float32),
            pltpu.VMEM((BPW,), jnp.float32),
            pltpu.VMEM((BPW,), jnp.float32),
            pltpu.SemaphoreType.DMA,
            pltpu.SemaphoreType.DMA,
        ],
        compiler_params=pltpu.CompilerParams(use_tc_tiling_on_sc=True,
                                             needs_layout_passes=False),
    )
    ceT, peT, cb, pb = sc_call(cids, pids, cembT, pembT, cbias, pbias)

    w1cT = W1[:D, :].T
    w1pT = W1[D:, :].T
    const = (0.4 * b3 + global_bias).reshape(1, 1)

    grid = (B // BLK,)
    out = pl.pallas_call(
        _dense_body,
        grid=grid,
        in_specs=[
            pl.BlockSpec((D, BLK), lambda i: (0, i)),
            pl.BlockSpec((D, BLK), lambda i: (0, i)),
            pl.BlockSpec((1, BLK), lambda i: (0, i)),
            pl.BlockSpec((1, BLK), lambda i: (0, i)),
            pl.BlockSpec((64, D), lambda i: (0, 0)),
            pl.BlockSpec((64, D), lambda i: (0, 0)),
            pl.BlockSpec((64, 1), lambda i: (0, 0)),
            pl.BlockSpec((D, 64), lambda i: (0, 0)),
            pl.BlockSpec((D, 1), lambda i: (0, 0)),
            pl.BlockSpec((D, 1), lambda i: (0, 0)),
            pl.BlockSpec((1, 1), lambda i: (0, 0)),
        ],
        out_specs=pl.BlockSpec((1, BLK), lambda i: (0, i)),
        out_shape=jax.ShapeDtypeStruct((1, B), jnp.float32),
    )(ceT, peT, cb.reshape(1, B), pb.reshape(1, B), w1cT, w1pT,
      b1.reshape(64, 1), W2.T, b2.reshape(D, 1), W3, const)
    return out.reshape(B)


# speedup vs baseline: 1.4247x; 1.3719x over previous
import jax
import jax.numpy as jnp
from jax import lax
from jax.experimental import pallas as pl
from jax.experimental.pallas import tpu as pltpu
from jax.experimental.pallas import tpu_sc as plsc

B = 16384
D = 32
N_ROWS = 1000000
NC = 2
NS = 16
NW = NC * NS
BPW = B // NW
G = 4

BLK = 2048


def _gather_one_table(idx_v, tabT, biasT, bufs, bbufs, stageT, stageB, sem):
    r0 = lax.iota(jnp.int32, 16)
    z16 = r0 * 0
    lane0 = r0 < 1

    def issue_q(idv16, q, buf, bbuf):
        for l in range(G):
            w0 = pl.multiple_of((idv16[G * q + l] >> 7) * 128, 128)
            pltpu.async_copy(tabT.at[:, pl.ds(w0, 128)], buf.at[l], sem)
            pltpu.async_copy(biasT.at[:, pl.ds(w0, 128)], bbuf.at[l], sem)

    def drain_q(buf, bbuf):
        for l in range(G):
            pltpu.make_async_copy(tabT.at[:, pl.ds(0, 128)],
                                  buf.at[l], sem).wait()
            pltpu.make_async_copy(biasT.at[:, pl.ds(0, 128)],
                                  bbuf.at[l], sem).wait()

    def extract_q(k, idv16, q, buf, bbuf):
        for l in range(G):
            colv = jnp.full((16,), idv16[G * q + l] & 127, jnp.int32)
            g0 = plsc.load_gather(buf.at[l], [r0, colv])
            g1 = plsc.load_gather(buf.at[l], [r0 + 16, colv])
            jcol = jnp.full((16,), k * 16 + G * q + l, jnp.int32)
            plsc.store_scatter(stageT, [r0, jcol], g0)
            plsc.store_scatter(stageT, [r0 + 16, jcol], g1)
            bv = plsc.load_gather(bbuf.at[l], [z16, colv])
            plsc.store_scatter(stageB, [z16, jcol], bv, mask=lane0)

    b0, b1, b2, b3 = bufs
    c0, c1, c2, c3 = bbufs
    idv_first = idx_v[pl.ds(0, 16)]
    issue_q(idv_first, 0, b0, c0)
    issue_q(idv_first, 1, b1, c1)
    issue_q(idv_first, 2, b2, c2)

    def body(k, _):
        idvk = idx_v[pl.ds(k * 16, 16)]
        idvn = idx_v[pl.ds(jnp.minimum((k + 1) * 16, BPW - 16), 16)]
        issue_q(idvk, 3, b3, c3)
        drain_q(b0, c0)
        extract_q(k, idvk, 0, b0, c0)
        issue_q(idvn, 0, b0, c0)
        drain_q(b1, c1)
        extract_q(k, idvk, 1, b1, c1)
        issue_q(idvn, 1, b1, c1)
        drain_q(b2, c2)
        extract_q(k, idvk, 2, b2, c2)
        issue_q(idvn, 2, b2, c2)
        drain_q(b3, c3)
        extract_q(k, idvk, 3, b3, c3)
        return 0

    lax.fori_loop(0, BPW // 16, body, 0)
    drain_q(b0, c0)
    drain_q(b1, c1)
    drain_q(b2, c2)


def _sc_gather(cust_ids, prod_ids, cembT, pembT, cbiasT, pbiasT,
               ceT_out, peT_out, cb_out, pb_out,
               cidx_v, pidx_v, b0, b1, b2, b3, c0, c1, c2, c3,
               stageT, stageB, sem):
    wid = lax.axis_index("s") * NC + lax.axis_index("c")
    base = wid * BPW
    pltpu.sync_copy(cust_ids.at[pl.ds(base, BPW)], cidx_v)
    pltpu.sync_copy(prod_ids.at[pl.ds(base, BPW)], pidx_v)

    bufs = (b0, b1, b2, b3)
    bbufs = (c0, c1, c2, c3)
    _gather_one_table(cidx_v, cembT, cbiasT, bufs, bbufs, stageT, stageB,
                      sem)
    pltpu.sync_copy(stageT, ceT_out.at[:, pl.ds(base, BPW)])
    pltpu.sync_copy(stageB, cb_out.at[:, pl.ds(base, BPW)])
    _gather_one_table(pidx_v, pembT, pbiasT, bufs, bbufs, stageT, stageB,
                      sem)
    pltpu.sync_copy(stageT, peT_out.at[:, pl.ds(base, BPW)])
    pltpu.sync_copy(stageB, pb_out.at[:, pl.ds(base, BPW)])


def _dense_body(ceT_ref, peT_ref, cb_ref, pb_ref, w1cT_ref, w1pT_ref, b1_ref,
                w2T_ref, b2_ref, w3_ref, const_ref, out_ref):
    ceT = ceT_ref[...]
    peT = peT_ref[...]
    mf = jnp.sum(ceT * peT, axis=0, keepdims=True)
    h1 = jnp.maximum(
        jnp.dot(w1cT_ref[...], ceT, preferred_element_type=jnp.float32)
        + jnp.dot(w1pT_ref[...], peT, preferred_element_type=jnp.float32)
        + b1_ref[...], 0.0)
    h2 = jnp.maximum(
        jnp.dot(w2T_ref[...], h1, preferred_element_type=jnp.float32)
        + b2_ref[...], 0.0)
    mlp = jnp.sum(h2 * w3_ref[...], axis=0, keepdims=True)
    logit = (0.6 * mf + 0.4 * mlp + cb_ref[...] + pb_ref[...]
             + const_ref[...])
    out_ref[...] = jax.nn.sigmoid(logit)


def kernel(customer_ids, product_ids, cust_emb, prod_emb, cust_bias,
           prod_bias, global_bias, W1, b1, W2, b2, W3, b3):
    cids = customer_ids.astype(jnp.int32)
    pids = product_ids.astype(jnp.int32)
    cembT = cust_emb.T
    pembT = prod_emb.T
    cbiasT = cust_bias.reshape(1, N_ROWS)
    pbiasT = prod_bias.reshape(1, N_ROWS)

    mesh = plsc.VectorSubcoreMesh(
        core_axis_name="c", subcore_axis_name="s",
        num_cores=NC, num_subcores=NS)
    sc_call = pl.kernel(
        _sc_gather,
        out_type=[
            jax.ShapeDtypeStruct((D, B), jnp.float32),
            jax.ShapeDtypeStruct((D, B), jnp.float32),
            jax.ShapeDtypeStruct((1, B), jnp.float32),
            jax.ShapeDtypeStruct((1, B), jnp.float32),
        ],
        mesh=mesh,
        scratch_types=[
            pltpu.VMEM((BPW,), jnp.int32),
            pltpu.VMEM((BPW,), jnp.int32),
            pltpu.VMEM((G, D, 128), jnp.float32),
            pltpu.VMEM((G, D, 128), jnp.float32),
            pltpu.VMEM((G, D, 128), jnp.float32),
            pltpu.VMEM((G, D, 128), jnp.float32),
            pltpu.VMEM((G, 1, 128), jnp.float32),
            pltpu.VMEM((G, 1, 128), jnp.float32),
            pltpu.VMEM((G, 1, 128), jnp.float32),
            pltpu.VMEM((G, 1, 128), jnp.float32),
            pltpu.VMEM((D, BPW), jnp.float32),
            pltpu.VMEM((1, BPW), jnp.float32),
            pltpu.SemaphoreType.DMA,
        ],
        compiler_params=pltpu.CompilerParams(use_tc_tiling_on_sc=True,
                                             needs_layout_passes=False),
    )
    ceT, peT, cb, pb = sc_call(cids, pids, cembT, pembT, cbiasT, pbiasT)

    w1cT = W1[:D, :].T
    w1pT = W1[D:, :].T
    const = (0.4 * b3 + global_bias).reshape(1, 1)

    grid = (B // BLK,)
    out = pl.pallas_call(
        _dense_body,
        grid=grid,
        in_specs=[
            pl.BlockSpec((D, BLK), lambda i: (0, i)),
            pl.BlockSpec((D, BLK), lambda i: (0, i)),
            pl.BlockSpec((1, BLK), lambda i: (0, i)),
            pl.BlockSpec((1, BLK), lambda i: (0, i)),
            pl.BlockSpec((64, D), lambda i: (0, 0)),
            pl.BlockSpec((64, D), lambda i: (0, 0)),
            pl.BlockSpec((64, 1), lambda i: (0, 0)),
            pl.BlockSpec((D, 64), lambda i: (0, 0)),
            pl.BlockSpec((D, 1), lambda i: (0, 0)),
            pl.BlockSpec((D, 1), lambda i: (0, 0)),
            pl.BlockSpec((1, 1), lambda i: (0, 0)),
        ],
        out_specs=pl.BlockSpec((1, BLK), lambda i: (0, i)),
        out_shape=jax.ShapeDtypeStruct((1, B), jnp.float32),
    )(ceT, peT, cb, pb, w1cT, w1pT,
      b1.reshape(64, 1), W2.T, b2.reshape(D, 1), W3, const)
    return out.reshape(B)
